# baseline (speedup 1.0000x reference)
import functools

import jax
import jax.numpy as jnp
from jax import lax
from jax.experimental import pallas as pl
from jax.experimental.pallas import tpu as pltpu
from jax.experimental.pallas import tpu_sc as plsc

B = 16384
D = 32
W = 128

_NC, _NS = 2, 16
_NW = _NC * _NS
_BPW = B // _NW
_CHUNK = 128
_NCH = _BPW // _CHUNK


def _sc_gather(user_idx2d, item_idx2d, usr, itm):
  mesh = plsc.VectorSubcoreMesh(core_axis_name="c", subcore_axis_name="s")

  out_t = jax.ShapeDtypeStruct((B, W), jnp.float32)
  buf_t = pltpu.VMEM((_CHUNK, W), jnp.float32)

  @functools.partial(
      pl.kernel,
      mesh=mesh,
      out_type=[out_t, out_t],
      scratch_types=[
          pltpu.VMEM((_NCH, _CHUNK), jnp.int32),
          pltpu.VMEM((_NCH, _CHUNK), jnp.int32),
          buf_t, buf_t, buf_t, buf_t,
          pltpu.SemaphoreType.DMA,
      ],
  )
  def k(uidx_hbm, iidx_hbm, usr_hbm, itm_hbm, u_o, i_o,
        uidx_v, iidx_v, ub0, ub1, ib0, ib1, sem):
    wid = lax.axis_index("s") * _NC + lax.axis_index("c")
    base = wid * _BPW
    crow = wid * _NCH
    pltpu.sync_copy(uidx_hbm.at[pl.ds(crow, _NCH)], uidx_v)
    pltpu.sync_copy(iidx_hbm.at[pl.ds(crow, _NCH)], iidx_v)
    ubufs = (ub0, ub1)
    ibufs = (ib0, ib1)
    prev = None
    for c in range(_NCH):
      cu = pltpu.async_copy(usr_hbm.at[uidx_v.at[c]], ubufs[c % 2], sem)
      ci = pltpu.async_copy(itm_hbm.at[iidx_v.at[c]], ibufs[c % 2], sem)
      if prev is not None:
        pcu, pci, pc = prev
        pcu.wait()
        pci.wait()
        out_sl = pl.ds(base + pc * _CHUNK, _CHUNK)
        pltpu.sync_copy(ubufs[pc % 2], u_o.at[out_sl])
        pltpu.sync_copy(ibufs[pc % 2], i_o.at[out_sl])
      prev = (cu, ci, c)
    pcu, pci, pc = prev
    pcu.wait()
    pci.wait()
    out_sl = pl.ds(base + pc * _CHUNK, _CHUNK)
    pltpu.sync_copy(ubufs[pc % 2], u_o.at[out_sl])
    pltpu.sync_copy(ibufs[pc % 2], i_o.at[out_sl])

  return k(user_idx2d, item_idx2d, usr, itm)


def _tc_dense_body(u_r, i_r, gw_r, gb_r, w1a_r, w1b_r, b1_r,
                   w2_r, b2_r, w3_r, b3_r, w4_r, b4_r, out_r):
  u = u_r[...]
  it = i_r[...]
  ug = u[:, :D]
  um = u[:, D:2 * D]
  ig = it[:, :D]
  im = it[:, D:2 * D]
  gmf_logit = jnp.sum(ug * ig * gw_r[...], axis=1, keepdims=True) + gb_r[0, 0]
  h = jnp.maximum(
      jnp.dot(um, w1a_r[...], preferred_element_type=jnp.float32)
      + jnp.dot(im, w1b_r[...], preferred_element_type=jnp.float32)
      + b1_r[...], 0.0)
  h = jnp.maximum(
      jnp.dot(h, w2_r[...], preferred_element_type=jnp.float32) + b2_r[...],
      0.0)
  h = jnp.maximum(
      jnp.dot(h, w3_r[...], preferred_element_type=jnp.float32) + b3_r[...],
      0.0)
  mlp_logit = jnp.sum(h * w4_r[...], axis=1, keepdims=True) + b4_r[0, 0]
  out_r[...] = 0.5 * (jax.nn.sigmoid(gmf_logit) + jax.nn.sigmoid(mlp_logit))


def kernel(user_indices, item_indices, emb_user_gmf, emb_user_mlp,
           emb_item_gmf, emb_item_mlp, gmf_w, gmf_b, w1, b1, w2, b2, w3, b3,
           w4, b4):
  uidx = jnp.asarray(user_indices, jnp.int32).reshape(B // _CHUNK, _CHUNK)
  iidx = jnp.asarray(item_indices, jnp.int32).reshape(B // _CHUNK, _CHUNK)

  uzero = jnp.zeros((emb_user_gmf.shape[0], W - 2 * D), jnp.float32)
  izero = jnp.zeros((emb_item_gmf.shape[0], W - 2 * D), jnp.float32)
  usr = jnp.concatenate([emb_user_gmf, emb_user_mlp, uzero], axis=1)
  itm = jnp.concatenate([emb_item_gmf, emb_item_mlp, izero], axis=1)

  urows, irows = _sc_gather(uidx, iidx, usr, itm)

  gw = gmf_w.reshape(1, D)
  gb = gmf_b.reshape(1, 1)
  w1a = w1[:D]
  w1b = w1[D:]
  b1r = b1.reshape(1, -1)
  b2r = b2.reshape(1, -1)
  b3r = b3.reshape(1, -1)
  w4r = w4.reshape(1, -1)
  b4r = b4.reshape(1, 1)

  blk = 4096
  grid = B // blk

  def row_spec():
    return pl.BlockSpec((blk, W), lambda i: (i, 0))

  def full_spec(shape):
    return pl.BlockSpec(shape, lambda i: tuple(0 for _ in shape))

  out = pl.pallas_call(
      _tc_dense_body,
      grid=(grid,),
      in_specs=[
          row_spec(), row_spec(),
          full_spec(gw.shape), full_spec(gb.shape),
          full_spec(w1a.shape), full_spec(w1b.shape), full_spec(b1r.shape),
          full_spec(w2.shape), full_spec(b2r.shape),
          full_spec(w3.shape), full_spec(b3r.shape),
          full_spec(w4r.shape), full_spec(b4r.shape),
      ],
      out_specs=pl.BlockSpec((blk, 1), lambda i: (i, 0)),
      out_shape=jax.ShapeDtypeStruct((B, 1), jnp.float32),
  )(urows, irows, gw, gb, w1a, w1b, b1r, w2, b2r, w3, b3r, w4r, b4r)
  return out
